# Initial kernel scaffold; baseline (speedup 1.0000x reference)
import jax
import jax.numpy as jnp
from jax.experimental import pallas as pl


def kernel(inputs, w, b):
    raise NotImplementedError("write your pallas kernel here")


# speedup vs baseline: 3.2808x; 3.2808x over previous
import functools

import jax
import jax.numpy as jnp
from jax import lax
from jax.experimental import pallas as pl
from jax.experimental.pallas import tpu as pltpu
from jax.experimental.pallas import tpu_sc as plsc

_VOCAB = 100000
_EMBED = 128
_NUM_IDX = 4096 * 50

_NC = 2
_NS = 16
_NW = _NC * _NS

_PER_W = _NUM_IDX // _NW
_GCH = 128
_GPC = 5
_ROWS = _GCH * _GPC
_NCHUNK = _PER_W // _ROWS


@functools.partial(
    pl.kernel,
    out_type=jax.ShapeDtypeStruct((_NUM_IDX, _EMBED), jnp.float32),
    mesh=plsc.VectorSubcoreMesh(core_axis_name="c", subcore_axis_name="s"),
    scratch_types=[
        pltpu.VMEM((_ROWS,), jnp.int32),
        pltpu.VMEM((_ROWS, _EMBED), jnp.float32),
        pltpu.SemaphoreType.DMA,
    ],
)
def _gather_kernel(table_hbm, idx_hbm, out_hbm, idx_v, rows_v, sem):
    wid = lax.axis_index("s") * _NC + lax.axis_index("c")
    base = wid * _PER_W

    def body(i, carry):
        off = base + i * _ROWS
        pltpu.sync_copy(idx_hbm.at[pl.ds(off, _ROWS)], idx_v)
        copies = [
            pltpu.async_copy(
                table_hbm.at[idx_v.at[pl.ds(j * _GCH, _GCH)]],
                rows_v.at[pl.ds(j * _GCH, _GCH)],
                sem,
            )
            for j in range(_GPC)
        ]
        for c in copies:
            c.wait()
        pltpu.sync_copy(rows_v, out_hbm.at[pl.ds(off, _ROWS)])
        return carry

    lax.fori_loop(0, _NCHUNK, body, 0)


def kernel(inputs, w, b):
    idx = jnp.reshape(inputs, (_NUM_IDX,)).astype(jnp.int32)
    out = _gather_kernel(w, idx)
    return jnp.reshape(out, (*inputs.shape, _EMBED))


# speedup vs baseline: 3.3418x; 1.0186x over previous
import functools

import jax
import jax.numpy as jnp
from jax import lax
from jax.experimental import pallas as pl
from jax.experimental.pallas import tpu as pltpu
from jax.experimental.pallas import tpu_sc as plsc

_VOCAB = 100000
_EMBED = 128
_NUM_IDX = 4096 * 50

_NC = 2
_NS = 16
_NW = _NC * _NS

_PER_W = _NUM_IDX // _NW
_GCH = 128
_KPC = 2
_ROWS = _GCH * _KPC
_NCHUNK = _PER_W // _ROWS


@functools.partial(
    pl.kernel,
    out_type=jax.ShapeDtypeStruct((_NUM_IDX, _EMBED), jnp.float32),
    mesh=plsc.VectorSubcoreMesh(core_axis_name="c", subcore_axis_name="s"),
    scratch_types=[
        pltpu.VMEM((_PER_W,), jnp.int32),
        pltpu.VMEM((_ROWS, _EMBED), jnp.float32),
        pltpu.VMEM((_ROWS, _EMBED), jnp.float32),
        pltpu.SemaphoreType.DMA,
        pltpu.SemaphoreType.DMA,
    ],
)
def _gather_kernel(table_hbm, idx_hbm, out_hbm, idx_v, buf0, buf1, sem_g, sem_w):
    wid = lax.axis_index("s") * _NC + lax.axis_index("c")
    base = wid * _PER_W
    pltpu.sync_copy(idx_hbm.at[pl.ds(base, _PER_W)], idx_v)

    bufs = (buf0, buf1)

    def fire_gather(c):
        return [
            pltpu.async_copy(
                table_hbm.at[idx_v.at[pl.ds(c * _ROWS + k * _GCH, _GCH)]],
                bufs[c % 2].at[pl.ds(k * _GCH, _GCH)],
                sem_g,
            )
            for k in range(_KPC)
        ]

    gathers = [fire_gather(0)]
    writes = []
    for c in range(_NCHUNK):
        if c >= 1:
            writes[c - 1].wait()
        if c + 1 < _NCHUNK:
            gathers.append(fire_gather(c + 1))
        for g in gathers[c]:
            g.wait()
        writes.append(
            pltpu.async_copy(
                bufs[c % 2],
                out_hbm.at[pl.ds(base + c * _ROWS, _ROWS)],
                sem_w,
            )
        )
    writes[_NCHUNK - 1].wait()


def kernel(inputs, w, b):
    idx = jnp.reshape(inputs, (_NUM_IDX,)).astype(jnp.int32)
    out = _gather_kernel(w, idx)
    return jnp.reshape(out, (*inputs.shape, _EMBED))


# speedup vs baseline: 3.3518x; 1.0030x over previous
import functools

import jax
import jax.numpy as jnp
from jax import lax
from jax.experimental import pallas as pl
from jax.experimental.pallas import tpu as pltpu
from jax.experimental.pallas import tpu_sc as plsc

_VOCAB = 100000
_EMBED = 128
_NUM_IDX = 4096 * 50

_NC = 2
_NS = 16
_NW = _NC * _NS

_PER_W = _NUM_IDX // _NW
_GCH = 400
_KPC = 1
_ROWS = _GCH * _KPC
_NCHUNK = _PER_W // _ROWS


@functools.partial(
    pl.kernel,
    out_type=jax.ShapeDtypeStruct((_NUM_IDX, _EMBED), jnp.float32),
    mesh=plsc.VectorSubcoreMesh(core_axis_name="c", subcore_axis_name="s"),
    scratch_types=[
        pltpu.VMEM((_PER_W,), jnp.int32),
        pltpu.VMEM((_ROWS, _EMBED), jnp.float32),
        pltpu.VMEM((_ROWS, _EMBED), jnp.float32),
        pltpu.SemaphoreType.DMA,
        pltpu.SemaphoreType.DMA,
    ],
)
def _gather_kernel(table_hbm, idx_hbm, out_hbm, idx_v, buf0, buf1, sem_g, sem_w):
    wid = lax.axis_index("s") * _NC + lax.axis_index("c")
    base = wid * _PER_W
    pltpu.sync_copy(idx_hbm.at[pl.ds(base, _PER_W)], idx_v)

    bufs = (buf0, buf1)

    def fire_gather(c):
        return [
            pltpu.async_copy(
                table_hbm.at[idx_v.at[pl.ds(c * _ROWS + k * _GCH, _GCH)]],
                bufs[c % 2].at[pl.ds(k * _GCH, _GCH)],
                sem_g,
            )
            for k in range(_KPC)
        ]

    gathers = [fire_gather(0)]
    writes = []
    for c in range(_NCHUNK):
        if c >= 1:
            writes[c - 1].wait()
        if c + 1 < _NCHUNK:
            gathers.append(fire_gather(c + 1))
        for g in gathers[c]:
            g.wait()
        writes.append(
            pltpu.async_copy(
                bufs[c % 2],
                out_hbm.at[pl.ds(base + c * _ROWS, _ROWS)],
                sem_w,
            )
        )
    writes[_NCHUNK - 1].wait()


def kernel(inputs, w, b):
    idx = jnp.reshape(inputs, (_NUM_IDX,)).astype(jnp.int32)
    out = _gather_kernel(w, idx)
    return jnp.reshape(out, (*inputs.shape, _EMBED))


# speedup vs baseline: 5.7733x; 1.7225x over previous
import functools

import jax
import jax.numpy as jnp
from jax import lax
from jax.experimental import pallas as pl
from jax.experimental.pallas import tpu as pltpu
from jax.experimental.pallas import tpu_sc as plsc

_VOCAB = 100000
_EMBED = 128
_B = 4096
_L = 50

_NC = 2
_NS = 16
_NW = _NC * _NS

_B_PER_W = _B // _NW
_BPC = 8
_NCHUNK = _B_PER_W // _BPC


@functools.partial(
    pl.kernel,
    out_type=jax.ShapeDtypeStruct((_B, _L, _EMBED), jnp.float32),
    mesh=plsc.VectorSubcoreMesh(core_axis_name="c", subcore_axis_name="s"),
    scratch_types=[
        pltpu.VMEM((_B_PER_W, _L), jnp.int32),
        pltpu.VMEM((_BPC, _L, _EMBED), jnp.float32),
        pltpu.VMEM((_BPC, _L, _EMBED), jnp.float32),
        pltpu.SemaphoreType.DMA,
        pltpu.SemaphoreType.DMA,
    ],
)
def _gather_kernel(table_hbm, idx_hbm, out_hbm, idx_v, buf0, buf1, sem_g, sem_w):
    wid = lax.axis_index("s") * _NC + lax.axis_index("c")
    base = wid * _B_PER_W
    pltpu.sync_copy(idx_hbm.at[pl.ds(base, _B_PER_W)], idx_v)

    bufs = (buf0, buf1)

    def fire_gather(c):
        return [
            pltpu.async_copy(
                table_hbm.at[idx_v.at[c * _BPC + k]],
                bufs[c % 2].at[k],
                sem_g,
            )
            for k in range(_BPC)
        ]

    gathers = [fire_gather(0)]
    writes = []
    for c in range(_NCHUNK):
        if c >= 1:
            writes[c - 1].wait()
        if c + 1 < _NCHUNK:
            gathers.append(fire_gather(c + 1))
        for g in gathers[c]:
            g.wait()
        writes.append(
            pltpu.async_copy(
                bufs[c % 2],
                out_hbm.at[pl.ds(base + c * _BPC, _BPC)],
                sem_w,
            )
        )
    writes[_NCHUNK - 1].wait()


def kernel(inputs, w, b):
    return _gather_kernel(w, inputs.astype(jnp.int32))


# speedup vs baseline: 10.4336x; 1.8072x over previous
import functools

import jax
import jax.numpy as jnp
from jax import lax
from jax.experimental import pallas as pl
from jax.experimental.pallas import tpu as pltpu
from jax.experimental.pallas import tpu_sc as plsc

_VOCAB = 100000
_EMBED = 128
_B = 4096
_L = 50

_NC = 2
_NS = 16
_NW = _NC * _NS

_B_PER_W = _B // _NW
_LPC = 2
_NCHUNK = _L // _LPC


@functools.partial(
    pl.kernel,
    out_type=jax.ShapeDtypeStruct((_L, _B, _EMBED), jnp.float32),
    mesh=plsc.VectorSubcoreMesh(core_axis_name="c", subcore_axis_name="s"),
    scratch_types=[
        pltpu.VMEM((_L, _B_PER_W), jnp.int32),
        pltpu.VMEM((_LPC, _B_PER_W, _EMBED), jnp.float32),
        pltpu.VMEM((_LPC, _B_PER_W, _EMBED), jnp.float32),
        pltpu.SemaphoreType.DMA,
        pltpu.SemaphoreType.DMA,
    ],
)
def _gather_kernel(table_hbm, idx_hbm, out_hbm, idx_v, buf0, buf1, sem_g, sem_w):
    wid = lax.axis_index("s") * _NC + lax.axis_index("c")
    b0 = wid * _B_PER_W
    pltpu.sync_copy(idx_hbm.at[:, pl.ds(b0, _B_PER_W)], idx_v)

    bufs = (buf0, buf1)

    def fire_gather(c):
        return [
            pltpu.async_copy(
                table_hbm.at[idx_v.at[c * _LPC + k]],
                bufs[c % 2].at[k],
                sem_g,
            )
            for k in range(_LPC)
        ]

    gathers = [fire_gather(0)]
    writes = []
    for c in range(_NCHUNK):
        if c >= 1:
            writes[c - 1].wait()
        if c + 1 < _NCHUNK:
            gathers.append(fire_gather(c + 1))
        for g in gathers[c]:
            g.wait()
        writes.append(
            pltpu.async_copy(
                bufs[c % 2],
                out_hbm.at[pl.ds(c * _LPC, _LPC), pl.ds(b0, _B_PER_W)],
                sem_w,
            )
        )
    writes[_NCHUNK - 1].wait()


def kernel(inputs, w, b):
    idx_t = jnp.transpose(inputs).astype(jnp.int32)
    out = _gather_kernel(w, idx_t)
    return jnp.transpose(out, (1, 0, 2))


# speedup vs baseline: 10.4926x; 1.0057x over previous
import functools

import jax
import jax.numpy as jnp
from jax import lax
from jax.experimental import pallas as pl
from jax.experimental.pallas import tpu as pltpu
from jax.experimental.pallas import tpu_sc as plsc

_VOCAB = 100000
_EMBED = 128
_B = 4096
_L = 50

_NC = 2
_NS = 16

_LG = 2
_BG = 16
_L_PER_W = _L // _LG
_B_PER_W = _B // _BG
_PER_W = _L_PER_W * _B_PER_W
_NBUF = 3


@functools.partial(
    pl.kernel,
    out_type=jax.ShapeDtypeStruct((_L, _B, _EMBED), jnp.float32),
    mesh=plsc.VectorSubcoreMesh(core_axis_name="c", subcore_axis_name="s"),
    scratch_types=[
        pltpu.VMEM((_PER_W,), jnp.int32),
        pltpu.VMEM((_B_PER_W, _EMBED), jnp.float32),
        pltpu.VMEM((_B_PER_W, _EMBED), jnp.float32),
        pltpu.VMEM((_B_PER_W, _EMBED), jnp.float32),
        pltpu.SemaphoreType.DMA,
        pltpu.SemaphoreType.DMA,
    ],
)
def _gather_kernel(table_hbm, idx_hbm, out_hbm, idx_v, buf0, buf1, buf2, sem_g, sem_w):
    lg = lax.axis_index("c")
    bg = lax.axis_index("s")
    wid = lg * _BG + bg
    l0 = lg * _L_PER_W
    b0 = bg * _B_PER_W
    pltpu.sync_copy(idx_hbm.at[pl.ds(wid * _PER_W, _PER_W)], idx_v)

    bufs = (buf0, buf1, buf2)

    def fire_gather(c):
        return pltpu.async_copy(
            table_hbm.at[idx_v.at[pl.ds(c * _B_PER_W, _B_PER_W)]],
            bufs[c % _NBUF],
            sem_g,
        )

    def fire_write(c):
        return pltpu.async_copy(
            bufs[c % _NBUF], out_hbm.at[l0 + c, pl.ds(b0, _B_PER_W)], sem_w
        )

    gathers = [fire_gather(0), fire_gather(1)]
    writes = []
    for c in range(_L_PER_W):
        if c >= 2:
            writes[c - 2].wait()
        if c + 2 < _L_PER_W:
            gathers.append(fire_gather(c + 2))
        gathers[c].wait()
        writes.append(fire_write(c))
    writes[_L_PER_W - 2].wait()
    writes[_L_PER_W - 1].wait()


def kernel(inputs, w, b):
    idx_t = jnp.transpose(inputs).astype(jnp.int32)
    idx_w = jnp.transpose(
        jnp.reshape(idx_t, (_LG, _L_PER_W, _BG, _B_PER_W)), (0, 2, 1, 3)
    )
    out = _gather_kernel(w, jnp.reshape(idx_w, (_L * _B,)))
    return jnp.transpose(out, (1, 0, 2))
